# baseline (speedup 1.0000x reference)
import functools

import jax
import jax.numpy as jnp
from jax import lax
from jax.experimental import pallas as pl
from jax.experimental.pallas import tpu as pltpu
from jax.experimental.pallas import tpu_sc as plsc

EMBED = 16
CHUNK = 128
BC = 1024
RPU = BC // CHUNK
NC, NS = 2, 16
NW = NC * NS


@functools.lru_cache(maxsize=None)
def _build(batch, fields):
    cpf = batch // BC
    units_per_w = fields * cpf // NW
    mesh = plsc.VectorSubcoreMesh(core_axis_name="c", subcore_axis_name="s")

    @functools.partial(
        pl.kernel,
        mesh=mesh,
        out_type=jax.ShapeDtypeStruct((fields, EMBED, batch), jnp.float32),
        scratch_types=[
            pltpu.VMEM((BC, fields), jnp.int32),
            pltpu.VMEM((2, BC), jnp.int32),
            pltpu.VMEM((2, BC, EMBED), jnp.float32),
            pltpu.VMEM((2, EMBED, BC), jnp.float32),
            pltpu.SemaphoreType.DMA,
            pltpu.SemaphoreType.DMA,
        ],
        compiler_params=pltpu.CompilerParams(
            use_tc_tiling_on_sc=False, needs_layout_passes=False),
    )
    def emb(idx_hbm, table_hbm, out_hbm,
            xstage_v, idx_v, rows_v, vals_v, gsem, ssem):
        wid = lax.axis_index("s") * NC + lax.axis_index("c")
        iota = lax.iota(jnp.int32, 16)
        iv = [vj * 16 + iota for vj in range(8)]

        def stage_and_fire(uidx, pp):
            gid = wid * units_per_w + uidx
            f = gid // cpf
            c = gid % cpf
            pltpu.sync_copy(idx_hbm.at[pl.ds(c * BC, BC), :], xstage_v)
            cid = jnp.zeros((16,), jnp.int32) + f
            for k in range(BC // 16):
                idx_v[pp, pl.ds(k * 16, 16)] = plsc.load_gather(
                    xstage_v, [k * 16 + iota, cid])
            for r in range(RPU):
                pltpu.async_copy(
                    table_hbm.at[idx_v.at[pp, pl.ds(r * CHUNK, CHUNK)]],
                    rows_v.at[pp, pl.ds(r * CHUNK, CHUNK), :],
                    gsem)

        stage_and_fire(0, 0)

        def unit(u, carry):
            pc = lax.rem(u, 2)
            gid = wid * units_per_w + u
            f = gid // cpf
            c = gid % cpf

            @pl.when(u >= 2)
            def _drain_stores():
                for e in range(EMBED):
                    pltpu.make_async_copy(
                        vals_v.at[pc, e],
                        out_hbm.at[0, 0, pl.ds(0, BC)],
                        ssem).wait()

            for r in range(RPU):
                pltpu.make_async_copy(
                    table_hbm.at[idx_v.at[pc, pl.ds(0, CHUNK)]],
                    rows_v.at[pc, pl.ds(r * CHUNK, CHUNK), :],
                    gsem).wait()

            @pl.when(u + 1 < units_per_w)
            def _prefetch():
                stage_and_fire(u + 1, 1 - pc)

            def vloop(vb, c3):
                base = vb * 128
                for e in range(EMBED):
                    cid = jnp.full((16,), e, jnp.int32)
                    for vj in range(8):
                        vec = plsc.load_gather(
                            rows_v.at[pc], [base + iv[vj], cid])
                        vals_v[pc, e, pl.ds(base + vj * 16, 16)] = vec
                return c3

            lax.fori_loop(0, BC // 128, vloop, 0)

            for e in range(EMBED):
                pltpu.async_copy(
                    vals_v.at[pc, e],
                    out_hbm.at[f, e, pl.ds(c * BC, BC)],
                    ssem)
            return carry

        lax.fori_loop(0, units_per_w, unit, 0)
        for q in range(2):
            for e in range(EMBED):
                pltpu.make_async_copy(
                    vals_v.at[q, e],
                    out_hbm.at[0, 0, pl.ds(0, BC)],
                    ssem).wait()

    return emb


def kernel(x, weight):
    batch, fields = x.shape
    o3 = _build(batch, fields)(x.astype(jnp.int32), weight)
    return o3.transpose(2, 0, 1)


# speedup vs baseline: 1.1098x; 1.1098x over previous
import functools

import jax
import jax.numpy as jnp
from jax import lax
from jax.experimental import pallas as pl
from jax.experimental.pallas import tpu as pltpu
from jax.experimental.pallas import tpu_sc as plsc

EMBED = 16
CHUNK = 128
BC = 1024
RPU = BC // CHUNK
NC, NS = 2, 16
NW = NC * NS


@functools.lru_cache(maxsize=None)
def _build(batch, fields):
    cpf = batch // BC
    units_per_w = fields * cpf // NW
    mesh = plsc.VectorSubcoreMesh(core_axis_name="c", subcore_axis_name="s")

    @functools.partial(
        pl.kernel,
        mesh=mesh,
        out_type=jax.ShapeDtypeStruct((fields, EMBED, batch), jnp.float32),
        scratch_types=[
            pltpu.VMEM((BC, fields), jnp.float32),
            pltpu.VMEM((2, BC), jnp.int32),
            pltpu.VMEM((2, BC, EMBED), jnp.float32),
            pltpu.VMEM((2, EMBED, BC), jnp.float32),
            pltpu.SemaphoreType.DMA,
            pltpu.SemaphoreType.DMA,
        ],
        compiler_params=pltpu.CompilerParams(
            use_tc_tiling_on_sc=False, needs_layout_passes=False),
    )
    def emb(idx_hbm, table_hbm, out_hbm,
            xstage_v, idx_v, rows_v, vals_v, gsem, ssem):
        wid = lax.axis_index("s") * NC + lax.axis_index("c")
        iota = lax.iota(jnp.int32, 16)
        cids = [jnp.full((16,), e, jnp.int32) for e in range(EMBED)]

        def stage_and_fire(uidx, pp):
            gid = wid * units_per_w + uidx
            f = gid // cpf
            c = gid % cpf
            pltpu.sync_copy(idx_hbm.at[pl.ds(c * BC, BC), :], xstage_v)
            cid = jnp.zeros((16,), jnp.int32) + f

            @plsc.parallel_loop(0, BC // 16, unroll=8)
            def _extract(k):
                idx_v[pp, pl.ds(k * 16, 16)] = plsc.bitcast(
                    plsc.load_gather(xstage_v, [k * 16 + iota, cid]),
                    jnp.int32)
            for r in range(RPU):
                pltpu.async_copy(
                    table_hbm.at[idx_v.at[pp, pl.ds(r * CHUNK, CHUNK)]],
                    rows_v.at[pp, pl.ds(r * CHUNK, CHUNK), :],
                    gsem)

        stage_and_fire(0, 0)

        def unit(u, carry):
            pc = lax.rem(u, 2)
            gid = wid * units_per_w + u
            f = gid // cpf
            c = gid % cpf

            @pl.when(u >= 2)
            def _drain_stores():
                for e in range(EMBED):
                    pltpu.make_async_copy(
                        vals_v.at[pc, e],
                        out_hbm.at[0, 0, pl.ds(0, BC)],
                        ssem).wait()

            for r in range(RPU):
                pltpu.make_async_copy(
                    table_hbm.at[idx_v.at[pc, pl.ds(0, CHUNK)]],
                    rows_v.at[pc, pl.ds(r * CHUNK, CHUNK), :],
                    gsem).wait()

            @pl.when(u + 1 < units_per_w)
            def _prefetch():
                stage_and_fire(u + 1, 1 - pc)

            @plsc.parallel_loop(0, BC // 16, unroll=4)
            def _transpose(j):
                rid = j * 16 + iota
                for e in range(EMBED):
                    vec = plsc.load_gather(rows_v.at[pc], [rid, cids[e]])
                    vals_v[pc, e, pl.ds(j * 16, 16)] = vec

            for e in range(EMBED):
                pltpu.async_copy(
                    vals_v.at[pc, e],
                    out_hbm.at[f, e, pl.ds(c * BC, BC)],
                    ssem)
            return carry

        lax.fori_loop(0, units_per_w, unit, 0)
        for q in range(2):
            for e in range(EMBED):
                pltpu.make_async_copy(
                    vals_v.at[q, e],
                    out_hbm.at[0, 0, pl.ds(0, BC)],
                    ssem).wait()

    return emb


def kernel(x, weight):
    batch, fields = x.shape
    xb = lax.bitcast_convert_type(x.astype(jnp.int32), jnp.float32)
    o3 = _build(batch, fields)(xb, weight)
    return o3.transpose(2, 0, 1)


# speedup vs baseline: 1.1820x; 1.0651x over previous
import functools

import jax
import jax.numpy as jnp
from jax import lax
from jax.experimental import pallas as pl
from jax.experimental.pallas import tpu as pltpu
from jax.experimental.pallas import tpu_sc as plsc

EMBED = 16
CHUNK = 128
BC = 1024
RPU = BC // CHUNK
NC, NS = 2, 16
NW = NC * NS


@functools.lru_cache(maxsize=None)
def _build(batch, fields):
    cpf = batch // BC
    units_per_w = fields * cpf // NW
    mesh = plsc.VectorSubcoreMesh(core_axis_name="c", subcore_axis_name="s")

    @functools.partial(
        pl.kernel,
        mesh=mesh,
        out_type=jax.ShapeDtypeStruct((fields, EMBED, batch), jnp.float32),
        scratch_types=[
            pltpu.VMEM((2, RPU, CHUNK), jnp.int32),
            pltpu.VMEM((2, BC, EMBED), jnp.float32),
            pltpu.VMEM((2, EMBED, BC), jnp.float32),
            pltpu.SemaphoreType.DMA,
            pltpu.SemaphoreType.DMA,
        ],
        compiler_params=pltpu.CompilerParams(
            use_tc_tiling_on_sc=False, needs_layout_passes=False),
    )
    def emb(idx_hbm, table_hbm, out_hbm,
            idx_v, rows_v, vals_v, gsem, ssem):
        wid = lax.axis_index("s") * NC + lax.axis_index("c")
        iota = lax.iota(jnp.int32, 16)
        cids = [jnp.full((16,), e, jnp.int32) for e in range(EMBED)]

        def stage_and_fire(uidx, pp):
            gid = wid * units_per_w + uidx
            f = gid // cpf
            c = gid % cpf
            pltpu.sync_copy(
                idx_hbm.at[f, pl.ds(c * RPU, RPU), :], idx_v.at[pp])
            for r in range(RPU):
                pltpu.async_copy(
                    table_hbm.at[idx_v.at[pp, r]],
                    rows_v.at[pp, pl.ds(r * CHUNK, CHUNK), :],
                    gsem)

        stage_and_fire(0, 0)

        def unit(u, carry):
            pc = lax.rem(u, 2)
            gid = wid * units_per_w + u
            f = gid // cpf
            c = gid % cpf

            @pl.when(u >= 2)
            def _drain_stores():
                for e in range(EMBED):
                    pltpu.make_async_copy(
                        vals_v.at[pc, e],
                        out_hbm.at[0, 0, pl.ds(0, BC)],
                        ssem).wait()

            for r in range(RPU):
                pltpu.make_async_copy(
                    table_hbm.at[idx_v.at[pc, 0]],
                    rows_v.at[pc, pl.ds(r * CHUNK, CHUNK), :],
                    gsem).wait()

            @pl.when(u + 1 < units_per_w)
            def _prefetch():
                stage_and_fire(u + 1, 1 - pc)

            @plsc.parallel_loop(0, BC // 16, unroll=4)
            def _transpose(j):
                rid = j * 16 + iota
                for e in range(EMBED):
                    vec = plsc.load_gather(rows_v.at[pc], [rid, cids[e]])
                    vals_v[pc, e, pl.ds(j * 16, 16)] = vec

            for e in range(EMBED):
                pltpu.async_copy(
                    vals_v.at[pc, e],
                    out_hbm.at[f, e, pl.ds(c * BC, BC)],
                    ssem)
            return carry

        lax.fori_loop(0, units_per_w, unit, 0)
        for q in range(2):
            for e in range(EMBED):
                pltpu.make_async_copy(
                    vals_v.at[q, e],
                    out_hbm.at[0, 0, pl.ds(0, BC)],
                    ssem).wait()

    return emb


def _tc_transpose_body(x_ref, o_ref):
    o_ref[...] = x_ref[...].T.reshape(o_ref.shape)


@functools.lru_cache(maxsize=None)
def _build_tc_transpose(batch, fields):
    blk = 2048
    return pl.pallas_call(
        _tc_transpose_body,
        grid=(batch // blk,),
        in_specs=[pl.BlockSpec((blk, fields), lambda i: (i, 0))],
        out_specs=pl.BlockSpec(
            (fields, blk // CHUNK, CHUNK), lambda i: (0, i, 0)),
        out_shape=jax.ShapeDtypeStruct(
            (fields, batch // CHUNK, CHUNK), jnp.int32),
    )


def kernel(x, weight):
    batch, fields = x.shape
    xt = _build_tc_transpose(batch, fields)(x.astype(jnp.int32))
    o3 = _build(batch, fields)(xt, weight)
    return o3.transpose(2, 0, 1)
